# baseline (device time: 1363197 ns/iter reference)
import jax
import jax.numpy as jnp
from jax import lax
from jax.experimental import pallas as pl
from jax.experimental.pallas import tpu as pltpu

N_DEV = 8
M = 4096
N_TOT = 8192
CHUNK_M = M // N_DEV
PIECE_N = N_TOT // 4
N_STEPS = 2 * N_DEV - 2
LANES = ((0, 0), (1, 2 * PIECE_N), (0, PIECE_N), (1, 3 * PIECE_N))


def kernel(x, w_mat, scale_x, scale_w):
    def body(x_ref, w_ref, sx_ref, sw_ref, out_ref,
             sl0, sl1, sl2, sl3, pc0, pc1, pc2, pc3,
             sems, copy_sems, credits):
        slots = (sl0, sl1, sl2, sl3)
        pcs = (pc0, pc1, pc2, pc3)
        my = lax.axis_index("i")
        left = lax.rem(my + N_DEV - 1, N_DEV)
        right = lax.rem(my + 1, N_DEV)
        send_to = (right, left)
        ack_to = (left, right)

        barrier = pltpu.get_barrier_semaphore()
        for nbr in (left, right):
            pl.semaphore_signal(barrier, inc=1, device_id=(nbr,),
                                device_id_type=pl.DeviceIdType.MESH)
        pl.semaphore_wait(barrier, 2)

        scale = sx_ref[0] * sw_ref[0]

        def partial(c, base):
            xa = x_ref[pl.ds(c * CHUNK_M, CHUNK_M), :]
            wa = w_ref[:, base:base + PIECE_N]
            acc = lax.dot_general(xa, wa, (((1,), (0,)), ((), ())),
                                  preferred_element_type=jnp.int32)
            return acc.astype(jnp.float32) * scale

        def store_out(li, k, c):
            cp = pltpu.make_async_copy(
                slots[li].at[k],
                out_ref.at[pl.ds(c * CHUNK_M, CHUNK_M),
                           pl.ds(LANES[li][1], PIECE_N)],
                copy_sems.at[li, k])
            cp.start()
            return cp

        def rs_chunk(dirn, s):
            if dirn == 0:
                return lax.rem(my + 2 * N_DEV - s - 1, N_DEV)
            return lax.rem(my + s + 1, N_DEV)

        def ag_chunk(dirn, t):
            if dirn == 0:
                return lax.rem(my + N_DEV - t, N_DEV)
            return lax.rem(my + t, N_DEV)

        for li, (dirn, base) in enumerate(LANES):
            slots[li][1] = partial(my, base)

        def make_rd(li, s):
            k = s % 2
            return pltpu.make_async_remote_copy(
                src_ref=slots[li].at[1 - k], dst_ref=slots[li].at[k],
                send_sem=sems.at[li, 0], recv_sem=sems.at[li, 1],
                device_id=(send_to[LANES[li][0]],),
                device_id_type=pl.DeviceIdType.MESH)

        rds = [None] * 4
        pend = [[None, None] for _ in range(4)]

        for s in range(N_STEPS + 1):
            k = s % 2
            for li, (dirn, base) in enumerate(LANES):
                d = s - 1
                if s >= 1:
                    rds[li].wait_recv()
                    if d < N_DEV - 1:
                        slots[li][1 - k] = slots[li][1 - k] + pcs[li][...]
                        if d == N_DEV - 2:
                            own = lax.rem(
                                my + (1 if dirn == 0 else N_DEV - 1), N_DEV)
                            pend[li][1 - k] = store_out(li, 1 - k, own)
                    else:
                        pend[li][1 - k] = store_out(
                            li, 1 - k, ag_chunk(dirn, d - (N_DEV - 1)))
                    rds[li].wait_send()
                    if pend[li][k] is not None:
                        pend[li][k].wait()
                        pend[li][k] = None
                    if s < N_STEPS:
                        pl.semaphore_signal(credits.at[li], inc=1,
                                            device_id=(ack_to[dirn],),
                                            device_id_type=pl.DeviceIdType.MESH)
                if s < N_STEPS:
                    if s >= 1:
                        pl.semaphore_wait(credits.at[li], 1)
                    rds[li] = make_rd(li, s)
                    rds[li].start()
            if s < N_DEV - 1:
                for li, (dirn, base) in enumerate(LANES):
                    pcs[li][...] = partial(rs_chunk(dirn, s), base)

        for li in range(4):
            if pend[li][1] is not None:
                pend[li][1].wait()

    slot_shape = pltpu.VMEM((2, CHUNK_M, PIECE_N), jnp.float32)
    pc_shape = pltpu.VMEM((CHUNK_M, PIECE_N), jnp.float32)
    return pl.pallas_call(
        body,
        out_shape=jax.ShapeDtypeStruct((M, N_TOT), jnp.float32),
        in_specs=[
            pl.BlockSpec(memory_space=pltpu.VMEM),
            pl.BlockSpec(memory_space=pltpu.VMEM),
            pl.BlockSpec(memory_space=pltpu.SMEM),
            pl.BlockSpec(memory_space=pltpu.SMEM),
        ],
        out_specs=pl.BlockSpec(memory_space=pl.ANY),
        scratch_shapes=[
            slot_shape, slot_shape, slot_shape, slot_shape,
            pc_shape, pc_shape, pc_shape, pc_shape,
            pltpu.SemaphoreType.DMA((4, 2)),
            pltpu.SemaphoreType.DMA((4, 2)),
            pltpu.SemaphoreType.REGULAR((4,)),
        ],
        compiler_params=pltpu.CompilerParams(
            collective_id=0,
            vmem_limit_bytes=64 * 1024 * 1024,
        ),
    )(x, w_mat, scale_x, scale_w)
